# baseline (speedup 1.0000x reference)
import jax
import jax.numpy as jnp
from jax import lax
from jax.experimental import pallas as pl
from jax.experimental.pallas import tpu as pltpu
from jax.experimental.pallas import tpu_sc as plsc

_D = 512


def _body(l1_hbm, l0_hbm, out_hbm, buf):
    tid = lax.axis_index("s")

    @pl.when(tid < 8)
    def _():
        pltpu.sync_copy(l1_hbm.at[pl.ds(tid * 8, 8)], buf)
        pltpu.sync_copy(buf, out_hbm.at[pl.ds(tid * 8, 8)])

    @pl.when(tid >= 8)
    def _():
        m = tid - 8
        pltpu.sync_copy(l0_hbm.at[pl.ds(m * 8, 8), 0], buf)
        pltpu.sync_copy(buf, out_hbm.at[pl.ds(64 + m * 8, 8)])


@jax.jit
def kernel(layer1, layer0):
    mesh = plsc.VectorSubcoreMesh(
        core_axis_name="c", subcore_axis_name="s", num_cores=1
    )
    f = pl.kernel(
        _body,
        out_type=jax.ShapeDtypeStruct((128, 2, _D), jnp.float32),
        mesh=mesh,
        scratch_types=[pltpu.VMEM((8, 2, _D), jnp.float32)],
    )
    l1_p = layer1.reshape(8192, 2, _D)
    l0_q = layer0.reshape(4096, 2, 2, _D)
    return f(l1_p, l0_q).reshape(256, _D)
